# baseline (speedup 1.0000x reference)
import functools

import jax
import jax.numpy as jnp
from jax import lax
from jax.experimental import pallas as pl
from jax.experimental.pallas import tpu as pltpu
from jax.experimental.pallas import tpu_sc as plsc

_VOCAB_TILE = 2048
_IDX_CHUNK = 128


def _cbow_pool_sc(x, emb_table):
    B, CTX = x.shape
    V, E = emb_table.shape
    info = plsc.get_sparse_core_info()
    NC, NS = info.num_cores, info.num_subcores
    NW = NC * NS
    n_tok = (B * CTX) // NW
    n_ch = n_tok // _IDX_CHUNK
    b_per_w = B // NW
    tok_per_row = 128 // E
    x_grp = x.reshape(NW, n_tok)
    emb_rs = jnp.concatenate(
        [lax.slice(emb_table, (k, 0), (V, E), (tok_per_row, 1))
         for k in range(tok_per_row)],
        axis=1,
    )

    mesh = plsc.VectorSubcoreMesh(core_axis_name="c", subcore_axis_name="s")

    @functools.partial(
        pl.kernel,
        mesh=mesh,
        out_type=jax.ShapeDtypeStruct((B, E), jnp.float32),
        scratch_types=[
            pltpu.VMEM((n_tok,), jnp.int32),
            pltpu.VMEM((n_tok,), jnp.int32),
            pltpu.VMEM((n_tok, 128), jnp.float32),
            pltpu.VMEM((b_per_w, E), jnp.float32),
            pltpu.SemaphoreType.DMA,
        ],
    )
    def pool(x_hbm, tab_hbm, out_hbm, xv, idx_v, rows_v, m_v, sem):
        wid = lax.axis_index("s") * NC + lax.axis_index("c")
        pltpu.sync_copy(x_hbm.at[wid], xv)
        for k in range(n_tok // 16):
            v16 = xv[pl.ds(k * 16, 16)]
            idx_v[pl.ds(k * 16, 16)] = lax.shift_right_logical(v16, 3)
        copies = [
            pltpu.async_copy(
                tab_hbm.at[idx_v.at[pl.ds(g * _IDX_CHUNK, _IDX_CHUNK)]],
                rows_v.at[pl.ds(g * _IDX_CHUNK, _IDX_CHUNK)],
                sem,
            )
            for g in range(n_ch)
        ]
        for c in copies:
            c.wait()

        scale = jnp.float32(1.0 / CTX)
        accs = [None] * b_per_w
        for grp in range(n_tok // 16):
            xg = xv[pl.ds(grp * 16, 16)]
            for l in range(16):
                tok = grp * 16 + l
                bi = tok // CTX
                off = (xg[l] & (tok_per_row - 1)) * E
                e16 = rows_v[tok, pl.ds(off, E)]
                accs[bi] = e16 if accs[bi] is None else accs[bi] + e16
        for bi in range(b_per_w):
            m_v[bi] = accs[bi] * scale
        pltpu.sync_copy(m_v, out_hbm.at[pl.ds(wid * b_per_w, b_per_w)])

    return pool(x_grp, emb_rs)


def _project_tc(m, W, b):
    B, E = m.shape
    V = W.shape[0]
    T = _VOCAB_TILE
    n_blk = -(-V // T)
    Wt = W.T
    b2 = b.reshape(1, V)

    def body(w_ref, m_ref, b_ref, o_ref):
        o_ref[...] = lax.dot_general(
            w_ref[...], m_ref[...],
            (((0,), (1,)), ((), ())),
            preferred_element_type=jnp.float32,
        ) + b_ref[...].T

    return pl.pallas_call(
        body,
        grid=(n_blk,),
        in_specs=[
            pl.BlockSpec((E, T), lambda i: (0, i)),
            pl.BlockSpec((B, E), lambda i: (0, 0)),
            pl.BlockSpec((1, T), lambda i: (0, i)),
        ],
        out_specs=pl.BlockSpec((T, B), lambda i: (i, 0)),
        out_shape=jax.ShapeDtypeStruct((V, B), jnp.float32),
    )(Wt, m, b2)


def kernel(x, emb_table, W, b):
    m = _cbow_pool_sc(x, emb_table)
    return _project_tc(m, W, b).T


# speedup vs baseline: 2.7231x; 2.7231x over previous
import functools

import jax
import jax.numpy as jnp
from jax import lax
from jax.experimental import pallas as pl
from jax.experimental.pallas import tpu as pltpu
from jax.experimental.pallas import tpu_sc as plsc

_VOCAB_TILE = 2048
_IDX_CHUNK = 128


def _cbow_pool_sc(x, emb_table):
    B, CTX = x.shape
    V, E = emb_table.shape
    info = plsc.get_sparse_core_info()
    NC, NS = info.num_cores, info.num_subcores
    NW = NC * NS
    n_tok = (B * CTX) // NW
    n_ch = n_tok // _IDX_CHUNK
    b_per_w = B // NW
    x_t = x.reshape(NW, b_per_w, CTX).transpose(0, 2, 1).reshape(NW, n_tok)
    emb_t = emb_table.T

    mesh = plsc.VectorSubcoreMesh(core_axis_name="c", subcore_axis_name="s")

    @functools.partial(
        pl.kernel,
        mesh=mesh,
        compiler_params=pltpu.CompilerParams(use_tc_tiling_on_sc=False),
        out_type=jax.ShapeDtypeStruct((E, B), jnp.float32),
        scratch_types=[
            pltpu.VMEM((n_tok,), jnp.int32),
            pltpu.VMEM((E * n_tok,), jnp.float32),
            pltpu.VMEM((E, b_per_w), jnp.float32),
            pltpu.SemaphoreType.DMA,
        ],
    )
    def pool(x_hbm, tab_hbm, out_hbm, xv, rows_v, m_v, sem):
        wid = lax.axis_index("s") * NC + lax.axis_index("c")
        pltpu.sync_copy(x_hbm.at[wid], xv)
        pending = []
        for e in range(E):
            new = [
                pltpu.async_copy(
                    tab_hbm.at[e].at[xv.at[pl.ds(g * _IDX_CHUNK, _IDX_CHUNK)]],
                    rows_v.at[pl.ds(e * n_tok + g * _IDX_CHUNK, _IDX_CHUNK)],
                    sem,
                )
                for g in range(n_ch)
            ]
            for c in pending:
                c.wait()
            pending = new
        for c in pending:
            c.wait()

        scale = jnp.float32(1.0 / CTX)
        n_bg = b_per_w // 16
        for e in range(E):
            for bg in range(n_bg):
                acc = None
                for c in range(CTX):
                    v = rows_v[pl.ds(e * n_tok + c * b_per_w + bg * 16, 16)]
                    acc = v if acc is None else acc + v
                m_v[e, pl.ds(bg * 16, 16)] = acc * scale
        pltpu.sync_copy(m_v, out_hbm.at[:, pl.ds(wid * b_per_w, b_per_w)])

    return pool(x_t, emb_t)


def _project_tc(mT, W, b):
    E, B = mT.shape
    V = W.shape[0]
    T = _VOCAB_TILE
    n_blk = -(-V // T)
    Wt = W.T
    b2 = b.reshape(1, V)

    def body(w_ref, m_ref, b_ref, o_ref):
        o_ref[...] = lax.dot_general(
            w_ref[...], m_ref[...],
            (((0,), (0,)), ((), ())),
            preferred_element_type=jnp.float32,
        ) + b_ref[...].T

    return pl.pallas_call(
        body,
        grid=(n_blk,),
        in_specs=[
            pl.BlockSpec((E, T), lambda i: (0, i)),
            pl.BlockSpec((E, B), lambda i: (0, 0)),
            pl.BlockSpec((1, T), lambda i: (0, i)),
        ],
        out_specs=pl.BlockSpec((T, B), lambda i: (i, 0)),
        out_shape=jax.ShapeDtypeStruct((V, B), jnp.float32),
    )(Wt, mT, b2)


def kernel(x, emb_table, W, b):
    mT = _cbow_pool_sc(x, emb_table)
    return _project_tc(mT, W, b).T


# speedup vs baseline: 2.7587x; 1.0131x over previous
import functools

import jax
import jax.numpy as jnp
from jax import lax
from jax.experimental import pallas as pl
from jax.experimental.pallas import tpu as pltpu
from jax.experimental.pallas import tpu_sc as plsc

_VOCAB_TILE = 2048
_IDX_CHUNK = 128


def _cbow_pool_sc(x, emb_table):
    B, CTX = x.shape
    V, E = emb_table.shape
    info = plsc.get_sparse_core_info()
    NC, NS = info.num_cores, info.num_subcores
    NW = NC * NS
    n_tok = (B * CTX) // NW
    n_ch = n_tok // _IDX_CHUNK
    b_per_w = B // NW
    x_t = x.reshape(NW, b_per_w, CTX).transpose(0, 2, 1).reshape(NW, n_tok)
    emb_t = emb_table.T

    mesh = plsc.VectorSubcoreMesh(core_axis_name="c", subcore_axis_name="s")

    @functools.partial(
        pl.kernel,
        mesh=mesh,
        compiler_params=pltpu.CompilerParams(use_tc_tiling_on_sc=False),
        out_type=jax.ShapeDtypeStruct((E, B), jnp.float32),
        scratch_types=[
            pltpu.VMEM((n_tok,), jnp.int32),
            pltpu.VMEM((E * n_tok,), jnp.float32),
            pltpu.VMEM((E, b_per_w), jnp.float32),
            pltpu.SemaphoreType.DMA,
        ],
    )
    def pool(x_hbm, tab_hbm, out_hbm, xv, rows_v, m_v, sem):
        wid = lax.axis_index("s") * NC + lax.axis_index("c")
        pltpu.sync_copy(x_hbm.at[wid], xv)
        waves = []
        for e in range(E):
            waves.append([
                pltpu.async_copy(
                    tab_hbm.at[e].at[xv.at[pl.ds(g * _IDX_CHUNK, _IDX_CHUNK)]],
                    rows_v.at[pl.ds(e * n_tok + g * _IDX_CHUNK, _IDX_CHUNK)],
                    sem,
                )
                for g in range(n_ch)
            ])
            if e >= 3:
                for c in waves[e - 3]:
                    c.wait()
        for wave in waves[E - 3:]:
            for c in wave:
                c.wait()

        scale = jnp.float32(1.0 / CTX)
        n_bg = b_per_w // 16
        for e in range(E):
            for bg in range(n_bg):
                acc = None
                for c in range(CTX):
                    v = rows_v[pl.ds(e * n_tok + c * b_per_w + bg * 16, 16)]
                    acc = v if acc is None else acc + v
                m_v[e, pl.ds(bg * 16, 16)] = acc * scale
        pltpu.sync_copy(m_v, out_hbm.at[:, pl.ds(wid * b_per_w, b_per_w)])

    return pool(x_t, emb_t)


def _project_tc(mT, W, b):
    E, B = mT.shape
    V = W.shape[0]
    T = _VOCAB_TILE
    n_blk = -(-V // T)
    Wt = W.T
    b2 = b.reshape(1, V)

    def body(w_ref, m_ref, b_ref, o_ref):
        o_ref[...] = lax.dot_general(
            w_ref[...], m_ref[...],
            (((0,), (0,)), ((), ())),
            preferred_element_type=jnp.float32,
        ) + b_ref[...].T

    return pl.pallas_call(
        body,
        grid=(n_blk,),
        in_specs=[
            pl.BlockSpec((E, T), lambda i: (0, i)),
            pl.BlockSpec((E, B), lambda i: (0, 0)),
            pl.BlockSpec((1, T), lambda i: (0, i)),
        ],
        out_specs=pl.BlockSpec((T, B), lambda i: (i, 0)),
        out_shape=jax.ShapeDtypeStruct((V, B), jnp.float32),
    )(Wt, mT, b2)


def kernel(x, emb_table, W, b):
    mT = _cbow_pool_sc(x, emb_table)
    return _project_tc(mT, W, b).T


# speedup vs baseline: 2.7649x; 1.0022x over previous
import functools

import jax
import jax.numpy as jnp
from jax import lax
from jax.experimental import pallas as pl
from jax.experimental.pallas import tpu as pltpu
from jax.experimental.pallas import tpu_sc as plsc

_VOCAB_TILE = 2048
_IDX_CHUNK = 640


def _cbow_pool_sc(x, emb_table):
    B, CTX = x.shape
    V, E = emb_table.shape
    info = plsc.get_sparse_core_info()
    NC, NS = info.num_cores, info.num_subcores
    NW = NC * NS
    n_tok = (B * CTX) // NW
    n_ch = n_tok // _IDX_CHUNK
    b_per_w = B // NW
    x_t = x.reshape(NW, b_per_w, CTX).transpose(0, 2, 1).reshape(NW, n_tok)
    emb_t = emb_table.T

    mesh = plsc.VectorSubcoreMesh(core_axis_name="c", subcore_axis_name="s")

    @functools.partial(
        pl.kernel,
        mesh=mesh,
        compiler_params=pltpu.CompilerParams(use_tc_tiling_on_sc=False),
        out_type=jax.ShapeDtypeStruct((E, B), jnp.float32),
        scratch_types=[
            pltpu.VMEM((n_tok,), jnp.int32),
            pltpu.VMEM((E * n_tok,), jnp.float32),
            pltpu.VMEM((E, b_per_w), jnp.float32),
            pltpu.SemaphoreType.DMA,
        ],
    )
    def pool(x_hbm, tab_hbm, out_hbm, xv, rows_v, m_v, sem):
        wid = lax.axis_index("s") * NC + lax.axis_index("c")
        pltpu.sync_copy(x_hbm.at[wid], xv)
        waves = []
        for e in range(E):
            waves.append([
                pltpu.async_copy(
                    tab_hbm.at[e].at[xv.at[pl.ds(g * _IDX_CHUNK, _IDX_CHUNK)]],
                    rows_v.at[pl.ds(e * n_tok + g * _IDX_CHUNK, _IDX_CHUNK)],
                    sem,
                )
                for g in range(n_ch)
            ])
            if e >= 3:
                for c in waves[e - 3]:
                    c.wait()
        for wave in waves[E - 3:]:
            for c in wave:
                c.wait()

        scale = jnp.float32(1.0 / CTX)
        n_bg = b_per_w // 16
        for e in range(E):
            for bg in range(n_bg):
                acc = None
                for c in range(CTX):
                    v = rows_v[pl.ds(e * n_tok + c * b_per_w + bg * 16, 16)]
                    acc = v if acc is None else acc + v
                m_v[e, pl.ds(bg * 16, 16)] = acc * scale
        pltpu.sync_copy(m_v, out_hbm.at[:, pl.ds(wid * b_per_w, b_per_w)])

    return pool(x_t, emb_t)


def _project_tc(mT, W, b):
    E, B = mT.shape
    V = W.shape[0]
    T = _VOCAB_TILE
    n_blk = -(-V // T)
    Wt = W.T
    b2 = b.reshape(1, V)

    def body(w_ref, m_ref, b_ref, o_ref):
        o_ref[...] = lax.dot_general(
            w_ref[...], m_ref[...],
            (((0,), (0,)), ((), ())),
            preferred_element_type=jnp.float32,
        ) + b_ref[...].T

    return pl.pallas_call(
        body,
        grid=(n_blk,),
        in_specs=[
            pl.BlockSpec((E, T), lambda i: (0, i)),
            pl.BlockSpec((E, B), lambda i: (0, 0)),
            pl.BlockSpec((1, T), lambda i: (0, i)),
        ],
        out_specs=pl.BlockSpec((T, B), lambda i: (i, 0)),
        out_shape=jax.ShapeDtypeStruct((V, B), jnp.float32),
    )(Wt, mT, b2)


def kernel(x, emb_table, W, b):
    mT = _cbow_pool_sc(x, emb_table)
    return _project_tc(mT, W, b).T
